# baseline (speedup 1.0000x reference)
import functools

import jax
import jax.numpy as jnp
from jax import lax
from jax.experimental import pallas as pl
from jax.experimental.pallas import tpu as pltpu
from jax.experimental.pallas import tpu_sc as plsc

LAMBDA_REGRESSION = 2.0

B, C, H, W = 16, 19, 128, 128
KSC = 4
HCH = 16
NCH = H // HCH
N_TASKS = KSC * C * NCH
NW = 32
TPW = (N_TASKS + NW - 1) // NW
LANES = 16
GRAN = HCH * W // LANES


def _tc_body(xi_ref, ti_ref, xr1_ref, tr1_ref, xr2_ref, tr2_ref,
             out_ref, acc_ref):
    b = pl.program_id(0)

    @pl.when(b == 0)
    def _init():
        for k in range(5):
            acc_ref[k] = 0.0

    ti = ti_ref[...]
    tgt = ti[:, :C]
    mask = (jnp.sum(ti, axis=1, keepdims=True) > 0.0).astype(jnp.float32)

    xi = xi_ref[...]
    log_x = jnp.maximum(jnp.log(xi), -100.0)
    log_1mx = jnp.maximum(jnp.log(1.0 - xi), -100.0)
    bce = -(tgt * log_x + (1.0 - tgt) * log_1mx)
    acc_ref[0] += jnp.sum(mask * bce)
    acc_ref[1] += jnp.sum(mask)

    rmask = (tgt > 0.0).astype(jnp.float32)
    acc_ref[2] += jnp.sum(rmask)

    @pl.when(b >= KSC)
    def _reg():
        d1 = jnp.abs(xr1_ref[...] - tr1_ref[...])
        acc_ref[3] += jnp.sum(rmask * d1)
        d2 = jnp.abs(xr2_ref[...] - tr2_ref[...])
        acc_ref[4] += jnp.sum(rmask * d2)

    @pl.when(b == B - 1)
    def _finish():
        for k in range(5):
            out_ref[k] = acc_ref[k]


def _sc_body(tgt_hbm, xr1_hbm, tr1_hbm, xr2_hbm, tr2_hbm,
             out_hbm, tgt_v, xr1_v, tr1_v, xr2_v, tr2_v, acc_v):
    wid = lax.axis_index("s") * 2 + lax.axis_index("c")
    acc_v[0] = jnp.zeros((LANES,), jnp.float32)
    acc_v[1] = jnp.zeros((LANES,), jnp.float32)

    def task_body(t, carry):
        task = wid * TPW + t

        @pl.when(task < N_TASKS)
        def _run():
            bb = task // (C * NCH)
            rem = task % (C * NCH)
            cc = rem // NCH
            h0 = (rem % NCH) * HCH
            pltpu.sync_copy(tgt_hbm.at[bb, cc, 0, pl.ds(h0, HCH)], tgt_v)
            pltpu.sync_copy(xr1_hbm.at[bb, cc, :, pl.ds(h0, HCH)], xr1_v)
            pltpu.sync_copy(tr1_hbm.at[bb, cc, :, pl.ds(h0, HCH)], tr1_v)
            pltpu.sync_copy(xr2_hbm.at[bb, cc, :, pl.ds(h0, HCH)], xr2_v)
            pltpu.sync_copy(tr2_hbm.at[bb, cc, :, pl.ds(h0, HCH)], tr2_v)

            def gran_body(g, gc):
                row = g // (W // LANES)
                col = (g % (W // LANES)) * LANES
                mf = jnp.where(tgt_v[row, pl.ds(col, LANES)] > 0.0, 1.0, 0.0)
                d1 = (jnp.abs(xr1_v[0, row, pl.ds(col, LANES)]
                              - tr1_v[0, row, pl.ds(col, LANES)])
                      + jnp.abs(xr1_v[1, row, pl.ds(col, LANES)]
                                - tr1_v[1, row, pl.ds(col, LANES)]))
                d2 = (jnp.abs(xr2_v[0, row, pl.ds(col, LANES)]
                              - tr2_v[0, row, pl.ds(col, LANES)])
                      + jnp.abs(xr2_v[1, row, pl.ds(col, LANES)]
                                - tr2_v[1, row, pl.ds(col, LANES)]))
                acc_v[0] += mf * d1
                acc_v[1] += mf * d2
                return gc

            lax.fori_loop(0, GRAN, gran_body, 0)

        return carry

    lax.fori_loop(0, TPW, task_body, 0)
    pltpu.sync_copy(acc_v, out_hbm.at[wid])


@functools.partial(jax.jit, static_argnames=("interpret",))
def kernel(x_intensity, x_reg1, x_reg2, target_intensity, target_reg1,
           target_reg2, target_scale, interpret=False):
    del target_scale

    spec1 = lambda c: pl.BlockSpec((1, c, 1, H, W), lambda b: (b, 0, 0, 0, 0))
    spec2 = pl.BlockSpec((1, C, 2, H, W),
                         lambda b: (jnp.maximum(b, KSC), 0, 0, 0, 0))

    tc_out = pl.pallas_call(
        _tc_body,
        grid=(B,),
        in_specs=[spec1(C), spec1(C + 1), spec2, spec2, spec2, spec2],
        out_specs=pl.BlockSpec(memory_space=pltpu.MemorySpace.SMEM),
        out_shape=jax.ShapeDtypeStruct((5,), jnp.float32),
        scratch_shapes=[pltpu.SMEM((5,), jnp.float32)],
        interpret=interpret,
    )(x_intensity, target_intensity, x_reg1, target_reg1, x_reg2, target_reg2)

    sc_out = pl.kernel(
        _sc_body,
        mesh=plsc.VectorSubcoreMesh(core_axis_name="c", subcore_axis_name="s"),
        out_type=jax.ShapeDtypeStruct((NW, 2, LANES), jnp.float32),
        scratch_types=[
            pltpu.VMEM((HCH, W), jnp.float32),
            pltpu.VMEM((2, HCH, W), jnp.float32),
            pltpu.VMEM((2, HCH, W), jnp.float32),
            pltpu.VMEM((2, HCH, W), jnp.float32),
            pltpu.VMEM((2, HCH, W), jnp.float32),
            pltpu.VMEM((2, LANES), jnp.float32),
        ],
    )(target_intensity, x_reg1, target_reg1, x_reg2, target_reg2)

    s_bce, s_mask, s_rm, s_l1_1, s_l1_2 = [tc_out[k] for k in range(5)]
    s_l1_1 = s_l1_1 + jnp.sum(sc_out[:, 0, :])
    s_l1_2 = s_l1_2 + jnp.sum(sc_out[:, 1, :])

    n_sel = jnp.float32(C) * s_mask
    n_reg = 2.0 * s_rm
    ce_loss = s_bce / n_sel
    scale = LAMBDA_REGRESSION / 1000.0 / jnp.float32(B)
    reg1_loss = scale * s_l1_1 / n_reg
    reg2_loss = scale * s_l1_2 / n_reg
    return (ce_loss, reg1_loss, reg2_loss)


# speedup vs baseline: 1.5538x; 1.5538x over previous
import functools

import jax
import jax.numpy as jnp
from jax import lax
from jax.experimental import pallas as pl
from jax.experimental.pallas import tpu as pltpu
from jax.experimental.pallas import tpu_sc as plsc

LAMBDA_REGRESSION = 2.0

B, C, H, W = 16, 19, 128, 128
KSC = 4
HCH = 16
NCH = H // HCH
N_TASKS = KSC * C * NCH
NW = 32
TPW = (N_TASKS + NW - 1) // NW
LANES = 16
GRAN = HCH * W // LANES


def _tc_body(xi_ref, ti_ref, xr1_ref, tr1_ref, xr2_ref, tr2_ref,
             out_ref, acc_ref):
    b = pl.program_id(0)

    @pl.when(b == 0)
    def _init():
        for k in range(5):
            acc_ref[k] = 0.0

    ti = ti_ref[...]
    tgt = ti[:, :C]
    mask = (jnp.sum(ti, axis=1, keepdims=True) > 0.0).astype(jnp.float32)

    xi = xi_ref[...]
    log_x = jnp.maximum(jnp.log(xi), -100.0)
    log_1mx = jnp.maximum(jnp.log(1.0 - xi), -100.0)
    bce = -(tgt * log_x + (1.0 - tgt) * log_1mx)
    acc_ref[0] += jnp.sum(mask * bce)
    acc_ref[1] += jnp.sum(mask)

    rmask = (tgt > 0.0).astype(jnp.float32)
    acc_ref[2] += jnp.sum(rmask)

    @pl.when(b >= KSC)
    def _reg():
        d1 = jnp.abs(xr1_ref[...] - tr1_ref[...])
        acc_ref[3] += jnp.sum(rmask * d1)
        d2 = jnp.abs(xr2_ref[...] - tr2_ref[...])
        acc_ref[4] += jnp.sum(rmask * d2)

    @pl.when(b == B - 1)
    def _finish():
        for k in range(5):
            out_ref[k] = acc_ref[k]


def _sc_body(tgt_hbm, xr1_hbm, tr1_hbm, xr2_hbm, tr2_hbm,
             out_hbm, tgt_v, xr1_v, tr1_v, xr2_v, tr2_v, acc_v, sem0, sem1):
    wid = lax.axis_index("s") * 2 + lax.axis_index("c")
    sems = (sem0, sem1)

    def copies(t, nb):
        task = wid * TPW + t
        bb = task // (C * NCH)
        rem = task % (C * NCH)
        cc = rem // NCH
        h0 = (rem % NCH) * HCH
        sem = sems[nb]
        return [
            pltpu.make_async_copy(tgt_hbm.at[bb, cc, 0, pl.ds(h0, HCH)],
                                  tgt_v.at[nb], sem),
            pltpu.make_async_copy(xr1_hbm.at[bb, cc, :, pl.ds(h0, HCH)],
                                  xr1_v.at[nb], sem),
            pltpu.make_async_copy(tr1_hbm.at[bb, cc, :, pl.ds(h0, HCH)],
                                  tr1_v.at[nb], sem),
            pltpu.make_async_copy(xr2_hbm.at[bb, cc, :, pl.ds(h0, HCH)],
                                  xr2_v.at[nb], sem),
            pltpu.make_async_copy(tr2_hbm.at[bb, cc, :, pl.ds(h0, HCH)],
                                  tr2_v.at[nb], sem),
        ]

    def compute(nb, s1, s2):
        def row_body(r, carry):
            cs1, cs2 = carry
            for gc in range(W // LANES):
                col = gc * LANES
                mf = jnp.where(tgt_v[nb, r, pl.ds(col, LANES)] > 0.0, 1.0, 0.0)
                d1 = (jnp.abs(xr1_v[nb, 0, r, pl.ds(col, LANES)]
                              - tr1_v[nb, 0, r, pl.ds(col, LANES)])
                      + jnp.abs(xr1_v[nb, 1, r, pl.ds(col, LANES)]
                                - tr1_v[nb, 1, r, pl.ds(col, LANES)]))
                d2 = (jnp.abs(xr2_v[nb, 0, r, pl.ds(col, LANES)]
                              - tr2_v[nb, 0, r, pl.ds(col, LANES)])
                      + jnp.abs(xr2_v[nb, 1, r, pl.ds(col, LANES)]
                                - tr2_v[nb, 1, r, pl.ds(col, LANES)]))
                cs1 = cs1 + mf * d1
                cs2 = cs2 + mf * d2
            return (cs1, cs2)

        return lax.fori_loop(0, HCH, row_body, (s1, s2))

    s1 = jnp.zeros((LANES,), jnp.float32)
    s2 = jnp.zeros((LANES,), jnp.float32)
    for c in copies(0, 0):
        c.start()
    for t in range(TPW):
        nb = t % 2
        if t + 1 < TPW:
            for c in copies(t + 1, 1 - nb):
                c.start()
        for c in copies(t, nb):
            c.wait()
        s1, s2 = compute(nb, s1, s2)
    acc_v[0] = s1
    acc_v[1] = s2
    pltpu.sync_copy(acc_v, out_hbm.at[wid])


@functools.partial(jax.jit, static_argnames=("interpret",))
def kernel(x_intensity, x_reg1, x_reg2, target_intensity, target_reg1,
           target_reg2, target_scale, interpret=False):
    del target_scale

    spec1 = lambda c: pl.BlockSpec((1, c, 1, H, W), lambda b: (b, 0, 0, 0, 0))
    spec2 = pl.BlockSpec((1, C, 2, H, W),
                         lambda b: (jnp.maximum(b, KSC), 0, 0, 0, 0))

    tc_out = pl.pallas_call(
        _tc_body,
        grid=(B,),
        in_specs=[spec1(C), spec1(C + 1), spec2, spec2, spec2, spec2],
        out_specs=pl.BlockSpec(memory_space=pltpu.MemorySpace.SMEM),
        out_shape=jax.ShapeDtypeStruct((5,), jnp.float32),
        scratch_shapes=[pltpu.SMEM((5,), jnp.float32)],
        interpret=interpret,
    )(x_intensity, target_intensity, x_reg1, target_reg1, x_reg2, target_reg2)

    sc_out = pl.kernel(
        _sc_body,
        mesh=plsc.VectorSubcoreMesh(core_axis_name="c", subcore_axis_name="s"),
        out_type=jax.ShapeDtypeStruct((NW, 2, LANES), jnp.float32),
        scratch_types=[
            pltpu.VMEM((2, HCH, W), jnp.float32),
            pltpu.VMEM((2, 2, HCH, W), jnp.float32),
            pltpu.VMEM((2, 2, HCH, W), jnp.float32),
            pltpu.VMEM((2, 2, HCH, W), jnp.float32),
            pltpu.VMEM((2, 2, HCH, W), jnp.float32),
            pltpu.VMEM((2, LANES), jnp.float32),
            pltpu.SemaphoreType.DMA,
            pltpu.SemaphoreType.DMA,
        ],
    )(target_intensity, x_reg1, target_reg1, x_reg2, target_reg2)

    s_bce, s_mask, s_rm, s_l1_1, s_l1_2 = [tc_out[k] for k in range(5)]
    s_l1_1 = s_l1_1 + jnp.sum(sc_out[:, 0, :])
    s_l1_2 = s_l1_2 + jnp.sum(sc_out[:, 1, :])

    n_sel = jnp.float32(C) * s_mask
    n_reg = 2.0 * s_rm
    ce_loss = s_bce / n_sel
    scale = LAMBDA_REGRESSION / 1000.0 / jnp.float32(B)
    reg1_loss = scale * s_l1_1 / n_reg
    reg2_loss = scale * s_l1_2 / n_reg
    return (ce_loss, reg1_loss, reg2_loss)
